# baseline (speedup 1.0000x reference)
import functools

import jax
import jax.numpy as jnp
from jax import lax
from jax.experimental import pallas as pl
from jax.experimental.pallas import tpu as pltpu
from jax.experimental.pallas import tpu_sc as plsc

_NC = 2
_NS = 16
_L = 16
_NW = _NC * _NS


@functools.partial(jax.jit, static_argnames=("batch", "embed"))
def _mf_call(row_id, col_id, row_emb, row_bias, col_emb, col_bias, gb8,
             batch, embed):
    bpw = batch // _NW
    n_grp = bpw // _L

    mesh = plsc.VectorSubcoreMesh(
        core_axis_name="c", subcore_axis_name="s",
        num_cores=_NC, num_subcores=_NS)

    @functools.partial(
        pl.kernel,
        out_type=jax.ShapeDtypeStruct((batch,), jnp.float32),
        mesh=mesh,
        compiler_params=pltpu.CompilerParams(
            needs_layout_passes=False, use_tc_tiling_on_sc=False),
        scratch_types=[
            pltpu.VMEM((bpw,), jnp.int32),
            pltpu.VMEM((bpw,), jnp.int32),
            pltpu.VMEM((bpw, embed), jnp.float32),
            pltpu.VMEM((bpw, embed), jnp.float32),
            pltpu.VMEM((bpw,), jnp.float32),
            pltpu.VMEM((bpw,), jnp.float32),
            pltpu.VMEM((_L,), jnp.float32),
            pltpu.VMEM((bpw,), jnp.float32),
            pltpu.SemaphoreType.DMA,
            pltpu.SemaphoreType.DMA,
            pltpu.SemaphoreType.DMA,
            pltpu.SemaphoreType.DMA,
        ],
    )
    def mf_kernel(row_id_hbm, col_id_hbm, row_emb_hbm, row_bias_hbm,
                  col_emb_hbm, col_bias_hbm, gb_hbm, out_hbm,
                  ridx_v, cidx_v, rrow_v, crow_v, rb_v, cb_v, gb_v, out_v,
                  sem0, sem1, sem2, sem3):
        wid = lax.axis_index("s") * _NC + lax.axis_index("c")
        base = wid * bpw

        pltpu.sync_copy(row_id_hbm.at[pl.ds(base, bpw)], ridx_v)
        pltpu.sync_copy(col_id_hbm.at[pl.ds(base, bpw)], cidx_v)

        d0 = pltpu.async_copy(row_emb_hbm.at[ridx_v], rrow_v, sem0)
        d1 = pltpu.async_copy(col_emb_hbm.at[cidx_v], crow_v, sem1)
        d2 = pltpu.async_copy(row_bias_hbm.at[ridx_v], rb_v, sem2)
        d3 = pltpu.async_copy(col_bias_hbm.at[cidx_v], cb_v, sem3)
        pltpu.sync_copy(gb_hbm, gb_v)
        d0.wait()
        d1.wait()
        d2.wait()
        d3.wait()

        gbv = gb_v[...]
        lane = lax.iota(jnp.int32, _L)

        def body(g, _):
            off = pl.multiple_of(g * _L, _L)
            bidx = off + lane
            acc = rb_v[pl.ds(off, _L)] + cb_v[pl.ds(off, _L)] + gbv
            for e in range(embed):
                eidx = jnp.full((_L,), e, jnp.int32)
                r = plsc.load_gather(rrow_v, [bidx, eidx])
                c = plsc.load_gather(crow_v, [bidx, eidx])
                acc = acc + r * c
            out_v[pl.ds(off, _L)] = acc
            return 0

        lax.fori_loop(0, n_grp, body, 0)

        pltpu.sync_copy(out_v, out_hbm.at[pl.ds(base, bpw)])

    return mf_kernel(row_id, col_id, row_emb, row_bias, col_emb, col_bias, gb8)


def kernel(row_id, col_id, row_emb_table, row_bias_table, col_emb_table,
           col_bias_table, global_bias):
    batch = row_id.shape[0]
    embed = row_emb_table.shape[1]
    gb8 = jnp.broadcast_to(jnp.reshape(global_bias, (1,)), (16,))
    out = _mf_call(row_id, col_id,
                   row_emb_table, jnp.reshape(row_bias_table, (-1,)),
                   col_emb_table, jnp.reshape(col_bias_table, (-1,)),
                   gb8, batch=batch, embed=embed)
    return out[:, None]
